# Initial kernel scaffold; baseline (speedup 1.0000x reference)
import jax
import jax.numpy as jnp
from jax.experimental import pallas as pl


def kernel(input, core1, core2, core3):
    raise NotImplementedError("write your pallas kernel here")


# speedup vs baseline: 5.2392x; 5.2392x over previous
import functools

import jax
import jax.numpy as jnp
from jax import lax
from jax.experimental import pallas as pl
from jax.experimental.pallas import tpu as pltpu
from jax.experimental.pallas import tpu_sc as plsc

M1, M2, M3 = 100, 100, 100
N1, N2, N3 = 4, 4, 4
RANK = 8
EMB_DIM = 64
NUM_PAIR = M2 * M3
ROW_W = RANK * N2 * N3

NC, NS = 2, 16
NW = NC * NS
CHUNK = 128
LANES = 16


def _pair_mm_body(a_ref, b_ref, o_ref):
    o_ref[...] = jnp.dot(a_ref[...], b_ref[...],
                         preferred_element_type=jnp.float32)


def _build_table23(core2, core3):
    a = core2.reshape(M2 * RANK * N2, RANK)
    b = core3.reshape(M3, RANK, N3).transpose(1, 0, 2).reshape(RANK, M3 * N3)
    p = pl.pallas_call(
        _pair_mm_body,
        out_shape=jax.ShapeDtypeStruct((M2 * RANK * N2, M3 * N3), jnp.float32),
    )(a, b)
    t = p.reshape(M2, RANK, N2, M3, N3).transpose(0, 3, 1, 2, 4)
    return t.reshape(NUM_PAIR, ROW_W)


def _sc_body(tok_per_w, idx_hbm, t23_hbm, c1_hbm, out_hbm,
             c1_v, idx_v, i23_v, rows_v, out_v, gsem):
    cid = lax.axis_index("c")
    sid = lax.axis_index("s")
    wid = sid * NC + cid
    base_w = wid * tok_per_w
    nchunk = tok_per_w // CHUNK
    ngroup = CHUNK // LANES

    pltpu.sync_copy(c1_hbm, c1_v)
    lanes = jnp.arange(LANES, dtype=jnp.int32)

    def chunk_body(c, carry):
        base = base_w + c * CHUNK
        pltpu.sync_copy(idx_hbm.at[pl.ds(base, CHUNK)], idx_v)

        def ig(g, carry2):
            v = idx_v[pl.ds(g * LANES, LANES)]
            i23_v[pl.ds(g * LANES, LANES)] = v % NUM_PAIR
            return carry2
        lax.fori_loop(0, ngroup, ig, 0)

        pltpu.async_copy(t23_hbm.at[i23_v], rows_v, gsem).wait()

        def cg(g, carry2):
            t0 = g * LANES
            tvec = t0 + lanes
            iv = idx_v[pl.ds(t0, LANES)]
            i1v = iv // NUM_PAIR
            c1v = [[plsc.load_gather(
                        c1_v, [i1v, jnp.full((LANES,), n1 * RANK + r1,
                                             jnp.int32)])
                    for r1 in range(RANK)] for n1 in range(N1)]
            for x in range(N2 * N3):
                t23v = [plsc.load_gather(
                            rows_v, [tvec, jnp.full((LANES,), r1 * N2 * N3 + x,
                                                    jnp.int32)])
                        for r1 in range(RANK)]
                for n1 in range(N1):
                    acc = c1v[n1][0] * t23v[0]
                    for r1 in range(1, RANK):
                        acc = acc + c1v[n1][r1] * t23v[r1]
                    plsc.store_scatter(
                        out_v,
                        [tvec, jnp.full((LANES,), n1 * N2 * N3 + x, jnp.int32)],
                        acc)
            return carry2
        lax.fori_loop(0, ngroup, cg, 0)

        pltpu.sync_copy(out_v, out_hbm.at[pl.ds(base, CHUNK)])
        return carry
    lax.fori_loop(0, nchunk, chunk_body, 0)


def _sc_lookup(flat_idx, table23, c1tab):
    b = flat_idx.shape[0]
    tok_per_w = b // NW
    mesh = plsc.VectorSubcoreMesh(core_axis_name="c", subcore_axis_name="s")
    return pl.kernel(
        functools.partial(_sc_body, tok_per_w),
        mesh=mesh,
        compiler_params=pltpu.CompilerParams(needs_layout_passes=False),
        out_type=jax.ShapeDtypeStruct((b, EMB_DIM), jnp.float32),
        scratch_types=[
            pltpu.VMEM((M1, N1 * RANK), jnp.float32),
            pltpu.VMEM((CHUNK,), jnp.int32),
            pltpu.VMEM((CHUNK,), jnp.int32),
            pltpu.VMEM((CHUNK, ROW_W), jnp.float32),
            pltpu.VMEM((CHUNK, EMB_DIM), jnp.float32),
            pltpu.SemaphoreType.DMA,
        ],
    )(flat_idx, table23, c1tab)


def kernel(input, core1, core2, core3):
    out_shape = (*input.shape, EMB_DIM)
    flat = input.reshape(-1).astype(jnp.int32)
    table23 = _build_table23(core2, core3)
    c1tab = core1.reshape(M1, N1 * RANK)
    out = _sc_lookup(flat, table23, c1tab)
    return out.reshape(out_shape)


# speedup vs baseline: 8.5020x; 1.6228x over previous
import functools

import jax
import jax.numpy as jnp
from jax import lax
from jax.experimental import pallas as pl
from jax.experimental.pallas import tpu as pltpu
from jax.experimental.pallas import tpu_sc as plsc

M1, M2, M3 = 100, 100, 100
N1, N2, N3 = 4, 4, 4
RANK = 8
EMB_DIM = 64
NUM_PAIR = M2 * M3
ROW_W = RANK * N2 * N3

NC, NS = 2, 16
NW = NC * NS
CHUNK = 128
LANES = 16
MM_G = 20


def _pair_mm_body(c3_ref, c2_ref, o_ref):
    for m in range(MM_G):
        o_ref[m] = jnp.dot(c3_ref[...], c2_ref[m],
                           preferred_element_type=jnp.float32)


def _build_table23(core2, core3):
    c3 = core3.reshape(M3, RANK, N3).transpose(0, 2, 1).reshape(M3 * N3, RANK)
    c2 = core2.reshape(M2, RANK * N2, RANK).transpose(0, 2, 1)
    t = pl.pallas_call(
        _pair_mm_body,
        grid=(M2 // MM_G,),
        in_specs=[
            pl.BlockSpec((M3 * N3, RANK), lambda i: (0, 0)),
            pl.BlockSpec((MM_G, RANK, RANK * N2), lambda i: (i, 0, 0)),
        ],
        out_specs=pl.BlockSpec((MM_G, M3 * N3, RANK * N2), lambda i: (i, 0, 0)),
        out_shape=jax.ShapeDtypeStruct((M2, M3 * N3, RANK * N2), jnp.float32),
    )(c3, c2)
    return t.reshape(NUM_PAIR, ROW_W)


def _t23_col(r1, n2, n3):
    return n3 * (RANK * N2) + r1 * N2 + n2


def _sc_body(tok_per_w, idx_hbm, t23_hbm, c1_hbm, out_hbm,
             c1_v, idx_v, i23_v, rows0, rows1, out0, out1, gsem0, gsem1):
    cid = lax.axis_index("c")
    sid = lax.axis_index("s")
    wid = sid * NC + cid
    base_w = wid * tok_per_w
    nch = tok_per_w // CHUNK

    pltpu.sync_copy(c1_hbm, c1_v)
    pltpu.sync_copy(idx_hbm.at[pl.ds(base_w, tok_per_w)], idx_v)
    lanes = jnp.arange(LANES, dtype=jnp.int32)

    def ig(g, carry):
        v = idx_v[pl.ds(g * LANES, LANES)]
        i23_v[pl.ds(g * LANES, LANES)] = v % NUM_PAIR
        return carry
    lax.fori_loop(0, tok_per_w // LANES, ig, 0)

    def fire(c, rows, sem):
        pltpu.async_copy(t23_hbm.at[i23_v.at[pl.ds(c * CHUNK, CHUNK)]],
                         rows, sem)

    def wait(rows, sem):
        pltpu.make_async_copy(t23_hbm.at[i23_v.at[pl.ds(0, CHUNK)]],
                              rows, sem).wait()

    def compute(c, rows, out):
        def cg(g, carry):
            t0 = g * LANES
            tvec = t0 + lanes
            iv = idx_v[pl.ds(c * CHUNK + t0, LANES)]
            i1v = iv // NUM_PAIR
            c1v = [[plsc.load_gather(
                        c1_v, [i1v, jnp.full((LANES,), n1 * RANK + r1,
                                             jnp.int32)])
                    for r1 in range(RANK)] for n1 in range(N1)]
            for n2 in range(N2):
                for n3 in range(N3):
                    t23v = [plsc.load_gather(
                                rows, [tvec, jnp.full((LANES,),
                                                      _t23_col(r1, n2, n3),
                                                      jnp.int32)])
                            for r1 in range(RANK)]
                    for n1 in range(N1):
                        acc = c1v[n1][0] * t23v[0]
                        for r1 in range(1, RANK):
                            acc = acc + c1v[n1][r1] * t23v[r1]
                        plsc.store_scatter(
                            out,
                            [tvec, jnp.full((LANES,),
                                            n1 * N2 * N3 + n2 * N3 + n3,
                                            jnp.int32)],
                            acc)
            return carry
        lax.fori_loop(0, CHUNK // LANES, cg, 0)
        pltpu.sync_copy(out, out_hbm.at[pl.ds(base_w + c * CHUNK, CHUNK)])

    fire(0, rows0, gsem0)

    def pair_body(i, carry):
        c0 = 2 * i
        fire(c0 + 1, rows1, gsem1)
        wait(rows0, gsem0)
        compute(c0, rows0, out0)

        @pl.when(i < nch // 2 - 1)
        def _():
            fire(c0 + 2, rows0, gsem0)
        wait(rows1, gsem1)
        compute(c0 + 1, rows1, out1)
        return carry
    lax.fori_loop(0, nch // 2, pair_body, 0)


def _sc_lookup(flat_idx, table23, c1tab):
    b = flat_idx.shape[0]
    tok_per_w = b // NW
    mesh = plsc.VectorSubcoreMesh(core_axis_name="c", subcore_axis_name="s")
    return pl.kernel(
        functools.partial(_sc_body, tok_per_w),
        mesh=mesh,
        compiler_params=pltpu.CompilerParams(needs_layout_passes=False),
        out_type=jax.ShapeDtypeStruct((b, EMB_DIM), jnp.float32),
        scratch_types=[
            pltpu.VMEM((M1, N1 * RANK), jnp.float32),
            pltpu.VMEM((tok_per_w,), jnp.int32),
            pltpu.VMEM((tok_per_w,), jnp.int32),
            pltpu.VMEM((CHUNK, ROW_W), jnp.float32),
            pltpu.VMEM((CHUNK, ROW_W), jnp.float32),
            pltpu.VMEM((CHUNK, EMB_DIM), jnp.float32),
            pltpu.VMEM((CHUNK, EMB_DIM), jnp.float32),
            pltpu.SemaphoreType.DMA,
            pltpu.SemaphoreType.DMA,
        ],
    )(flat_idx, table23, c1tab)


def kernel(input, core1, core2, core3):
    out_shape = (*input.shape, EMB_DIM)
    flat = input.reshape(-1).astype(jnp.int32)
    table23 = _build_table23(core2, core3)
    c1tab = core1.reshape(M1, N1 * RANK)
    out = _sc_lookup(flat, table23, c1tab)
    return out.reshape(out_shape)


# speedup vs baseline: 14.1326x; 1.6623x over previous
import functools

import jax
import jax.numpy as jnp
from jax import lax
from jax.experimental import pallas as pl
from jax.experimental.pallas import tpu as pltpu
from jax.experimental.pallas import tpu_sc as plsc

M1, M2, M3 = 100, 100, 100
N1, N2, N3 = 4, 4, 4
RANK = 8
EMB_DIM = 64
NUM_PAIR = M2 * M3
ROW_W = RANK * N2 * N3
N3_BLK = RANK * N2 + 4
ROW_P = 256
C1_W = N1 * RANK
C1_P = C1_W + 1

NC, NS = 2, 16
NW = NC * NS
CHUNK = 128
LANES = 16
MM_G = 20


def _pair_mm_body(c3_ref, c2_ref, o_ref):
    for m in range(MM_G):
        o_ref[m] = jnp.dot(c3_ref[...], c2_ref[m],
                           preferred_element_type=jnp.float32)


def _build_table23(core2, core3):
    c3 = core3.reshape(M3, RANK, N3).transpose(0, 2, 1).reshape(M3 * N3, RANK)
    c2 = core2.reshape(M2, RANK * N2, RANK).transpose(0, 2, 1)
    t = pl.pallas_call(
        _pair_mm_body,
        grid=(M2 // MM_G,),
        in_specs=[
            pl.BlockSpec((M3 * N3, RANK), lambda i: (0, 0)),
            pl.BlockSpec((MM_G, RANK, RANK * N2), lambda i: (i, 0, 0)),
        ],
        out_specs=pl.BlockSpec((MM_G, M3 * N3, RANK * N2), lambda i: (i, 0, 0)),
        out_shape=jax.ShapeDtypeStruct((M2, M3 * N3, RANK * N2), jnp.float32),
    )(c3, c2)
    t = t.reshape(NUM_PAIR, N3, RANK * N2)
    t = jnp.pad(t, ((0, 0), (0, 0), (0, N3_BLK - RANK * N2)))
    t = t.reshape(NUM_PAIR, N3 * N3_BLK)
    return jnp.pad(t, ((0, 0), (0, ROW_P - N3 * N3_BLK)))


def _sc_body(tok_per_w, idx_hbm, t23_hbm, c1_hbm, out_hbm,
             c1_v, idx_v, i23_v, rows0, rows1, out0, out1, gsem0, gsem1):
    cid = lax.axis_index("c")
    sid = lax.axis_index("s")
    wid = sid * NC + cid
    base_w = wid * tok_per_w
    nch = tok_per_w // CHUNK

    pltpu.sync_copy(c1_hbm, c1_v)
    pltpu.sync_copy(idx_hbm.at[pl.ds(base_w, tok_per_w)], idx_v)
    lanes = jnp.arange(LANES, dtype=jnp.int32)
    colbase = (lanes % N3) * N3_BLK + (lanes // N3)
    colv = [colbase + r1 * N2 for r1 in range(RANK)]

    def ig(g, carry):
        v = idx_v[pl.ds(g * LANES, LANES)]
        i23_v[pl.ds(g * LANES, LANES)] = v % NUM_PAIR
        return carry
    lax.fori_loop(0, tok_per_w // LANES, ig, 0)

    def fire(c, rows, sem):
        pltpu.async_copy(t23_hbm.at[i23_v.at[pl.ds(c * CHUNK, CHUNK)]],
                         rows, sem)

    def wait(rows, sem):
        pltpu.make_async_copy(t23_hbm.at[i23_v.at[pl.ds(0, CHUNK)]],
                              rows, sem).wait()

    def compute(c, rows, out):
        def cg(g, carry):
            t0 = g * LANES
            iv = idx_v[pl.ds(c * CHUNK + t0, LANES)]
            i1v = iv // NUM_PAIR
            for tg in range(LANES):
                ti = t0 + tg
                cbase = i1v[tg] * C1_P
                c1a = c1_v[pl.ds(cbase, LANES)]
                c1b = c1_v[pl.ds(cbase + LANES, LANES)]
                tiv = jnp.full((LANES,), ti, jnp.int32)
                t23v = [plsc.load_gather(rows, [tiv, colv[r1]])
                        for r1 in range(RANK)]
                obase = ti * EMB_DIM
                for n1 in range(N1):
                    p = []
                    for r1 in range(RANK):
                        j = n1 * RANK + r1
                        coef = c1a[j] if j < LANES else c1b[j - LANES]
                        p.append(coef * t23v[r1])
                    s0 = (p[0] + p[1]) + (p[2] + p[3])
                    s1 = (p[4] + p[5]) + (p[6] + p[7])
                    out[pl.ds(obase + n1 * LANES, LANES)] = s0 + s1
            return carry
        lax.fori_loop(0, CHUNK // LANES, cg, 0)
        pltpu.sync_copy(out, out_hbm.at[pl.ds((base_w + c * CHUNK) * EMB_DIM,
                                              CHUNK * EMB_DIM)])

    fire(0, rows0, gsem0)

    def pair_body(i, carry):
        c0 = 2 * i
        fire(c0 + 1, rows1, gsem1)
        wait(rows0, gsem0)
        compute(c0, rows0, out0)

        @pl.when(i < nch // 2 - 1)
        def _():
            fire(c0 + 2, rows0, gsem0)
        wait(rows1, gsem1)
        compute(c0 + 1, rows1, out1)
        return carry
    lax.fori_loop(0, nch // 2, pair_body, 0)


def _sc_lookup(flat_idx, table23, c1pad):
    b = flat_idx.shape[0]
    tok_per_w = b // NW
    mesh = plsc.VectorSubcoreMesh(core_axis_name="c", subcore_axis_name="s")
    return pl.kernel(
        functools.partial(_sc_body, tok_per_w),
        mesh=mesh,
        compiler_params=pltpu.CompilerParams(needs_layout_passes=False),
        out_type=jax.ShapeDtypeStruct((b * EMB_DIM,), jnp.float32),
        scratch_types=[
            pltpu.VMEM((M1 * C1_P,), jnp.float32),
            pltpu.VMEM((tok_per_w,), jnp.int32),
            pltpu.VMEM((tok_per_w,), jnp.int32),
            pltpu.VMEM((CHUNK, ROW_P), jnp.float32),
            pltpu.VMEM((CHUNK, ROW_P), jnp.float32),
            pltpu.VMEM((CHUNK * EMB_DIM,), jnp.float32),
            pltpu.VMEM((CHUNK * EMB_DIM,), jnp.float32),
            pltpu.SemaphoreType.DMA,
            pltpu.SemaphoreType.DMA,
        ],
    )(flat_idx, table23, c1pad)


def kernel(input, core1, core2, core3):
    out_shape = (*input.shape, EMB_DIM)
    flat = input.reshape(-1).astype(jnp.int32)
    table23 = _build_table23(core2, core3)
    c1pad = jnp.pad(core1.reshape(M1, C1_W), ((0, 0), (0, 1))).reshape(-1)
    out = _sc_lookup(flat, table23, c1pad)
    return out.reshape(out_shape)


# speedup vs baseline: 17.9368x; 1.2692x over previous
import functools

import jax
import jax.numpy as jnp
from jax import lax
from jax.experimental import pallas as pl
from jax.experimental.pallas import tpu as pltpu
from jax.experimental.pallas import tpu_sc as plsc

M1, M2, M3 = 100, 100, 100
N1, N2, N3 = 4, 4, 4
RANK = 8
EMB_DIM = 64
NUM_PAIR = M2 * M3
ROW_W = RANK * N2 * N3
ROW_P = ROW_W
C1_W = N1 * RANK
C1_P = C1_W + 1

NC, NS = 2, 16
NW = NC * NS
CHUNK = 128
LANES = 16
MM_G = 20


def _pair_mm_body(c3_ref, c2_ref, o_ref):
    for m in range(MM_G):
        p = jnp.dot(c3_ref[...], c2_ref[m],
                    preferred_element_type=jnp.float32)
        p = p.reshape(M3, N3, RANK * N2)
        o_ref[m] = jnp.concatenate(
            [p[:, n3, :] if n3 == 0 else
             jnp.roll(p[:, n3, :], N2 * n3, axis=-1) for n3 in range(N3)],
            axis=-1)


def _build_table23(core2, core3):
    c3 = core3.reshape(M3, RANK, N3).transpose(0, 2, 1).reshape(M3 * N3, RANK)
    c2 = core2.reshape(M2, RANK * N2, RANK).transpose(0, 2, 1)
    t = pl.pallas_call(
        _pair_mm_body,
        grid=(M2 // MM_G,),
        in_specs=[
            pl.BlockSpec((M3 * N3, RANK), lambda i: (0, 0)),
            pl.BlockSpec((MM_G, RANK, RANK * N2), lambda i: (i, 0, 0)),
        ],
        out_specs=pl.BlockSpec((MM_G, M3, ROW_W), lambda i: (i, 0, 0)),
        out_shape=jax.ShapeDtypeStruct((M2, M3, ROW_W), jnp.float32),
    )(c3, c2)
    return t.reshape(NUM_PAIR, ROW_P)


def _sc_body(tok_per_w, idx_hbm, t23_hbm, c1_hbm, out_hbm,
             c1_v, idx_v, i23_v, rows0, rows1, out0, out1,
             gsem0, gsem1, osem0, osem1):
    cid = lax.axis_index("c")
    sid = lax.axis_index("s")
    wid = sid * NC + cid
    base_w = wid * tok_per_w
    nch = tok_per_w // CHUNK

    pltpu.sync_copy(c1_hbm, c1_v)
    pltpu.sync_copy(idx_hbm.at[pl.ds(base_w, tok_per_w)], idx_v)
    lanes = jnp.arange(LANES, dtype=jnp.int32)
    n2v = lanes // N3
    n3v = lanes % N3
    colv = [n3v * (RANK * N2) + (r1 * N2 + n2v + N2 * n3v) % (RANK * N2)
            for r1 in range(RANK)]

    def ig(g, carry):
        v = idx_v[pl.ds(g * LANES, LANES)]
        i23_v[pl.ds(g * LANES, LANES)] = v % NUM_PAIR
        return carry
    lax.fori_loop(0, tok_per_w // LANES, ig, 0)

    def fire(c, rows, sem):
        pltpu.async_copy(t23_hbm.at[i23_v.at[pl.ds(c * CHUNK, CHUNK)]],
                         rows, sem)

    def wait(rows, sem):
        pltpu.make_async_copy(t23_hbm.at[i23_v.at[pl.ds(0, CHUNK)]],
                              rows, sem).wait()

    def wait_out(out, sem):
        pltpu.make_async_copy(out, out_hbm.at[pl.ds(0, CHUNK * EMB_DIM)],
                              sem).wait()

    def compute(c, rows, out, osem):
        def cg(g, carry):
            t0 = g * LANES
            iv = idx_v[pl.ds(c * CHUNK + t0, LANES)]
            i1v = iv // NUM_PAIR
            for tg in range(LANES):
                ti = t0 + tg
                cbase = i1v[tg] * C1_P
                c1a = c1_v[pl.ds(cbase, LANES)]
                c1b = c1_v[pl.ds(cbase + LANES, LANES)]
                tiv = jnp.full((LANES,), ti, jnp.int32)
                t23v = [plsc.load_gather(rows, [tiv, colv[r1]])
                        for r1 in range(RANK)]
                obase = ti * EMB_DIM
                for n1 in range(N1):
                    p = []
                    for r1 in range(RANK):
                        j = n1 * RANK + r1
                        coef = c1a[j] if j < LANES else c1b[j - LANES]
                        p.append(coef * t23v[r1])
                    s0 = (p[0] + p[1]) + (p[2] + p[3])
                    s1 = (p[4] + p[5]) + (p[6] + p[7])
                    out[pl.ds(obase + n1 * LANES, LANES)] = s0 + s1
            return carry
        lax.fori_loop(0, CHUNK // LANES, cg, 0)
        pltpu.async_copy(out,
                         out_hbm.at[pl.ds((base_w + c * CHUNK) * EMB_DIM,
                                          CHUNK * EMB_DIM)], osem)

    fire(0, rows0, gsem0)

    def pair_body(i, carry):
        c0 = 2 * i
        fire(c0 + 1, rows1, gsem1)
        wait(rows0, gsem0)

        @pl.when(i > 0)
        def _():
            wait_out(out0, osem0)
        compute(c0, rows0, out0, osem0)

        @pl.when(i < nch // 2 - 1)
        def _():
            fire(c0 + 2, rows0, gsem0)
        wait(rows1, gsem1)

        @pl.when(i > 0)
        def _():
            wait_out(out1, osem1)
        compute(c0 + 1, rows1, out1, osem1)
        return carry
    lax.fori_loop(0, nch // 2, pair_body, 0)
    wait_out(out0, osem0)
    wait_out(out1, osem1)


def _sc_lookup(flat_idx, table23, c1pad):
    b = flat_idx.shape[0]
    tok_per_w = b // NW
    mesh = plsc.VectorSubcoreMesh(core_axis_name="c", subcore_axis_name="s")
    return pl.kernel(
        functools.partial(_sc_body, tok_per_w),
        mesh=mesh,
        compiler_params=pltpu.CompilerParams(needs_layout_passes=False),
        out_type=jax.ShapeDtypeStruct((b * EMB_DIM,), jnp.float32),
        scratch_types=[
            pltpu.VMEM((M1 * C1_P,), jnp.float32),
            pltpu.VMEM((tok_per_w,), jnp.int32),
            pltpu.VMEM((tok_per_w,), jnp.int32),
            pltpu.VMEM((CHUNK, ROW_P), jnp.float32),
            pltpu.VMEM((CHUNK, ROW_P), jnp.float32),
            pltpu.VMEM((CHUNK * EMB_DIM,), jnp.float32),
            pltpu.VMEM((CHUNK * EMB_DIM,), jnp.float32),
            pltpu.SemaphoreType.DMA,
            pltpu.SemaphoreType.DMA,
            pltpu.SemaphoreType.DMA,
            pltpu.SemaphoreType.DMA,
        ],
    )(flat_idx, table23, c1pad)


def kernel(input, core1, core2, core3):
    out_shape = (*input.shape, EMB_DIM)
    flat = input.reshape(-1).astype(jnp.int32)
    table23 = _build_table23(core2, core3)
    c1pad = jnp.pad(core1.reshape(M1, C1_W), ((0, 0), (0, 1))).reshape(-1)
    out = _sc_lookup(flat, table23, c1pad)
    return out.reshape(out_shape)
